# baseline (speedup 1.0000x reference)
import functools
import math

import jax
import jax.numpy as jnp
from jax import lax
from jax.experimental import pallas as pl
from jax.experimental.pallas import tpu as pltpu
from jax.experimental.pallas import tpu_sc as plsc

VOCAB = 1000000
EMBED_DIM = 64
BATCH = 4096
HIST = 200

_NC = 2
_NS = 16
_NW = _NC * _NS
_BW = BATCH // _NW
_WIDE = 2 * EMBED_DIM
_L = 16
_BG = _BW // _L
_DG = EMBED_DIM // 8
_SCALE = math.sqrt(EMBED_DIM)


def _emb_body(idx_hbm, tab_hbm, out_hbm,
              idxh0, idxh1, rows0, rows1, tile0, tile1,
              isem0, isem1, gsem0, gsem1, osem0, osem1):
    wid = lax.axis_index("s") * _NC + lax.axis_index("c")
    iota16 = lax.iota(jnp.int32, 16)

    bufs = ((idxh0, rows0, tile0, isem0, gsem0, osem0),
            (idxh1, rows1, tile1, isem1, gsem1, osem1))

    def idx_src(h):
        return idx_hbm.at[pl.ds(h * BATCH + wid * _BW, _BW)]

    def start_idx(h, idxh, isem):
        pltpu.async_copy(idx_src(h), idxh, isem)

    def wait_idx(idxh, isem):
        pltpu.make_async_copy(idx_src(0), idxh, isem).wait()

    def start_gather(idxh, rows, gsem):
        pltpu.async_copy(tab_hbm.at[idxh], rows, gsem)

    def wait_gather(idxh, rows, gsem):
        pltpu.make_async_copy(tab_hbm.at[idxh], rows, gsem).wait()

    def transpose_scale(rows, tile):
        def body(dg, carry):
            colb = jnp.full((_L,), 0, jnp.int32) + dg * 8
            for ds in range(8):
                col16 = colb + ds
                for g in range(_BG):
                    row16 = iota16 + g * _L
                    v = plsc.load_gather(rows, [row16, col16])
                    tile[dg, ds, pl.ds(g * _L, _L)] = v * _SCALE
            return carry

        lax.fori_loop(0, _DG, body, 0)

    def start_out(h, tile, osem):
        pltpu.async_copy(tile, out_hbm.at[h, :, wid], osem)

    def wait_out(tile, osem):
        pltpu.make_async_copy(tile, out_hbm.at[0, :, wid], osem).wait()

    pltpu.sync_copy(idx_src(0), idxh0)
    start_gather(idxh0, rows0, gsem0)
    start_idx(1, idxh1, isem1)

    def step(j2, carry):
        for b in range(2):
            idxh_b, rows_b, tile_b, isem_b, gsem_b, osem_b = bufs[b]
            idxh_o, rows_o, tile_o, isem_o, gsem_o, osem_o = bufs[1 - b]
            cur = j2 * 2 + b

            @pl.when((cur >= 1) & (cur + 1 < HIST))
            def _():
                wait_out(tile_o, osem_o)

            @pl.when(cur + 1 < HIST)
            def _():
                wait_idx(idxh_o, isem_o)
                start_gather(idxh_o, rows_o, gsem_o)

            wait_gather(idxh_b, rows_b, gsem_b)

            @pl.when(cur + 2 < HIST)
            def _():
                start_idx(cur + 2, idxh_b, isem_b)

            transpose_scale(rows_b, tile_b)
            start_out(cur, tile_b, osem_b)
        return carry

    lax.fori_loop(0, HIST // 2, step, 0)

    wait_out(tile0, osem0)
    wait_out(tile1, osem1)


_mesh = plsc.VectorSubcoreMesh(core_axis_name="c", subcore_axis_name="s")

_emb = functools.partial(
    pl.kernel,
    mesh=_mesh,
    out_type=jax.ShapeDtypeStruct((HIST, _DG, _NW, 8, 128), jnp.float32),
    scratch_types=[
        pltpu.VMEM((_BW,), jnp.int32),
        pltpu.VMEM((_BW,), jnp.int32),
        pltpu.VMEM((_BW, _WIDE), jnp.float32),
        pltpu.VMEM((_BW, _WIDE), jnp.float32),
        pltpu.VMEM((_DG, 8, _BW), jnp.float32),
        pltpu.VMEM((_DG, 8, _BW), jnp.float32),
        pltpu.SemaphoreType.DMA,
        pltpu.SemaphoreType.DMA,
        pltpu.SemaphoreType.DMA,
        pltpu.SemaphoreType.DMA,
        pltpu.SemaphoreType.DMA,
        pltpu.SemaphoreType.DMA,
    ],
    compiler_params=pltpu.CompilerParams(
        use_tc_tiling_on_sc=False, needs_layout_passes=False),
)(_emb_body)


def kernel(x, table):
    flat_t = x.T.reshape(-1).astype(jnp.int32)
    wide = jnp.pad(table, ((0, 0), (0, EMBED_DIM)))
    out5 = _emb(flat_t, wide)
    return out5.transpose(2, 4, 0, 1, 3).reshape(BATCH, HIST, EMBED_DIM)


# speedup vs baseline: 1.7121x; 1.7121x over previous
import functools
import math

import jax
import jax.numpy as jnp
from jax import lax
from jax.experimental import pallas as pl
from jax.experimental.pallas import tpu as pltpu
from jax.experimental.pallas import tpu_sc as plsc

VOCAB = 1000000
EMBED_DIM = 64
BATCH = 4096
HIST = 200

_NC = 2
_NS = 16
_NW = _NC * _NS
_BW = BATCH // _NW
_WIDE = 2 * EMBED_DIM
_L = 16
_BG = _BW // _L
_DG = EMBED_DIM // 8
_SCALE = math.sqrt(EMBED_DIM)


def _emb_body(idx_hbm, tab_hbm, out_hbm,
              idxh0, idxh1, rows0, rows1, tile0, tile1,
              isem0, isem1, gsem0, gsem1, osem0, osem1):
    wid = lax.axis_index("s") * _NC + lax.axis_index("c")
    iota16 = lax.iota(jnp.int32, 16)

    bufs = ((idxh0, rows0, tile0, isem0, gsem0, osem0),
            (idxh1, rows1, tile1, isem1, gsem1, osem1))

    def idx_src(h):
        return idx_hbm.at[pl.ds(h * BATCH + wid * _BW, _BW)]

    def start_idx(h, idxh, isem):
        pltpu.async_copy(idx_src(h), idxh, isem)

    def wait_idx(idxh, isem):
        pltpu.make_async_copy(idx_src(0), idxh, isem).wait()

    def start_gather(idxh, rows, gsem):
        pltpu.async_copy(tab_hbm.at[idxh], rows, gsem)

    def wait_gather(idxh, rows, gsem):
        pltpu.make_async_copy(tab_hbm.at[idxh], rows, gsem).wait()

    def transpose_scale(rows, tile):
        def body(k, carry):
            dvec = (iota16 + k) & 63
            dgv = lax.shift_right_logical(dvec, 3)
            dsv = dvec & 7
            for g in range(_BG):
                rowv = iota16 + g * _L
                v = plsc.load_gather(rows, [rowv, dvec])
                plsc.store_scatter(tile, [dgv, dsv, rowv], v * _SCALE)
            return carry

        lax.fori_loop(0, EMBED_DIM, body, 0)

    def start_out(h, tile, osem):
        pltpu.async_copy(tile, out_hbm.at[h, :, wid], osem)

    def wait_out(tile, osem):
        pltpu.make_async_copy(tile, out_hbm.at[0, :, wid], osem).wait()

    pltpu.sync_copy(idx_src(0), idxh0)
    start_gather(idxh0, rows0, gsem0)
    start_idx(1, idxh1, isem1)

    def step(j2, carry):
        for b in range(2):
            idxh_b, rows_b, tile_b, isem_b, gsem_b, osem_b = bufs[b]
            idxh_o, rows_o, tile_o, isem_o, gsem_o, osem_o = bufs[1 - b]
            cur = j2 * 2 + b

            @pl.when((cur >= 1) & (cur + 1 < HIST))
            def _():
                wait_out(tile_o, osem_o)

            @pl.when(cur + 1 < HIST)
            def _():
                wait_idx(idxh_o, isem_o)
                start_gather(idxh_o, rows_o, gsem_o)

            wait_gather(idxh_b, rows_b, gsem_b)

            @pl.when(cur + 2 < HIST)
            def _():
                start_idx(cur + 2, idxh_b, isem_b)

            transpose_scale(rows_b, tile_b)
            start_out(cur, tile_b, osem_b)
        return carry

    lax.fori_loop(0, HIST // 2, step, 0)

    wait_out(tile0, osem0)
    wait_out(tile1, osem1)


_mesh = plsc.VectorSubcoreMesh(core_axis_name="c", subcore_axis_name="s")

_emb = functools.partial(
    pl.kernel,
    mesh=_mesh,
    out_type=jax.ShapeDtypeStruct((HIST, _DG, _NW, 8, 128), jnp.float32),
    scratch_types=[
        pltpu.VMEM((_BW,), jnp.int32),
        pltpu.VMEM((_BW,), jnp.int32),
        pltpu.VMEM((_BW, _WIDE), jnp.float32),
        pltpu.VMEM((_BW, _WIDE), jnp.float32),
        pltpu.VMEM((_DG, 8, _BW), jnp.float32),
        pltpu.VMEM((_DG, 8, _BW), jnp.float32),
        pltpu.SemaphoreType.DMA,
        pltpu.SemaphoreType.DMA,
        pltpu.SemaphoreType.DMA,
        pltpu.SemaphoreType.DMA,
        pltpu.SemaphoreType.DMA,
        pltpu.SemaphoreType.DMA,
    ],
    compiler_params=pltpu.CompilerParams(
        use_tc_tiling_on_sc=False, needs_layout_passes=False),
)(_emb_body)


def kernel(x, table):
    flat_t = x.T.reshape(-1).astype(jnp.int32)
    wide = jnp.pad(table, ((0, 0), (0, EMBED_DIM)))
    out5 = _emb(flat_t, wide)
    return out5.transpose(2, 4, 0, 1, 3).reshape(BATCH, HIST, EMBED_DIM)


# speedup vs baseline: 1.7280x; 1.0093x over previous
import functools
import math

import jax
import jax.numpy as jnp
from jax import lax
from jax.experimental import pallas as pl
from jax.experimental.pallas import tpu as pltpu
from jax.experimental.pallas import tpu_sc as plsc

VOCAB = 1000000
EMBED_DIM = 64
BATCH = 4096
HIST = 200

_NC = 2
_NS = 16
_NW = _NC * _NS
_BW = BATCH // _NW
_WIDE = 2 * EMBED_DIM
_L = 16
_BG = _BW // _L
_DG = EMBED_DIM // 8
_SCALE = math.sqrt(EMBED_DIM)


def _emb_body(idx_hbm, tab_hbm, out_hbm,
              idxh0, idxh1, rows0, rows1, tile0, tile1,
              isem0, isem1, gsem0, gsem1, osem0, osem1):
    wid = lax.axis_index("s") * _NC + lax.axis_index("c")
    iota16 = lax.iota(jnp.int32, 16)

    bufs = ((idxh0, rows0, tile0, isem0, gsem0, osem0),
            (idxh1, rows1, tile1, isem1, gsem1, osem1))

    def idx_src(h):
        return idx_hbm.at[pl.ds(h * BATCH + wid * _BW, _BW)]

    def start_idx(h, idxh, isem):
        pltpu.async_copy(idx_src(h), idxh, isem)

    def wait_idx(idxh, isem):
        pltpu.make_async_copy(idx_src(0), idxh, isem).wait()

    def start_gather(idxh, rows, gsem):
        pltpu.async_copy(tab_hbm.at[idxh], rows, gsem)

    def wait_gather(idxh, rows, gsem):
        pltpu.make_async_copy(tab_hbm.at[idxh], rows, gsem).wait()

    def transpose_scale(rows, tile):
        def body(k, carry):
            dvec = (iota16 + k) & 63
            dgv = lax.shift_right_logical(dvec, 3)
            dsv = dvec & 7
            for g in range(_BG):
                rowv = iota16 + g * _L
                v = plsc.load_gather(rows, [rowv, dvec])
                plsc.store_scatter(tile, [dgv, dsv, rowv], v * _SCALE)
            return carry

        lax.fori_loop(0, EMBED_DIM, body, 0, unroll=4)

    def start_out(h, tile, osem):
        pltpu.async_copy(tile, out_hbm.at[h, :, wid], osem)

    def wait_out(tile, osem):
        pltpu.make_async_copy(tile, out_hbm.at[0, :, wid], osem).wait()

    pltpu.sync_copy(idx_src(0), idxh0)
    start_gather(idxh0, rows0, gsem0)
    start_idx(1, idxh1, isem1)

    def step(j2, carry):
        for b in range(2):
            idxh_b, rows_b, tile_b, isem_b, gsem_b, osem_b = bufs[b]
            idxh_o, rows_o, tile_o, isem_o, gsem_o, osem_o = bufs[1 - b]
            cur = j2 * 2 + b

            @pl.when((cur >= 1) & (cur + 1 < HIST))
            def _():
                wait_out(tile_o, osem_o)

            @pl.when(cur + 1 < HIST)
            def _():
                wait_idx(idxh_o, isem_o)
                start_gather(idxh_o, rows_o, gsem_o)

            wait_gather(idxh_b, rows_b, gsem_b)

            @pl.when(cur + 2 < HIST)
            def _():
                start_idx(cur + 2, idxh_b, isem_b)

            transpose_scale(rows_b, tile_b)
            start_out(cur, tile_b, osem_b)
        return carry

    lax.fori_loop(0, HIST // 2, step, 0)

    wait_out(tile0, osem0)
    wait_out(tile1, osem1)


_mesh = plsc.VectorSubcoreMesh(core_axis_name="c", subcore_axis_name="s")

_emb = functools.partial(
    pl.kernel,
    mesh=_mesh,
    out_type=jax.ShapeDtypeStruct((HIST, _DG, _NW, 8, 128), jnp.float32),
    scratch_types=[
        pltpu.VMEM((_BW,), jnp.int32),
        pltpu.VMEM((_BW,), jnp.int32),
        pltpu.VMEM((_BW, _WIDE), jnp.float32),
        pltpu.VMEM((_BW, _WIDE), jnp.float32),
        pltpu.VMEM((_DG, 8, _BW), jnp.float32),
        pltpu.VMEM((_DG, 8, _BW), jnp.float32),
        pltpu.SemaphoreType.DMA,
        pltpu.SemaphoreType.DMA,
        pltpu.SemaphoreType.DMA,
        pltpu.SemaphoreType.DMA,
        pltpu.SemaphoreType.DMA,
        pltpu.SemaphoreType.DMA,
    ],
    compiler_params=pltpu.CompilerParams(
        use_tc_tiling_on_sc=False, needs_layout_passes=False),
)(_emb_body)


def kernel(x, table):
    flat_t = x.T.reshape(-1).astype(jnp.int32)
    wide = jnp.pad(table, ((0, 0), (0, EMBED_DIM)))
    out5 = _emb(flat_t, wide)
    return out5.transpose(2, 4, 0, 1, 3).reshape(BATCH, HIST, EMBED_DIM)
